# baseline (speedup 1.0000x reference)
import functools

import jax
import jax.numpy as jnp
from jax import lax
from jax.experimental import pallas as pl
from jax.experimental.pallas import tpu as pltpu
from jax.experimental.pallas import tpu_sc as plsc

_ROWS, _COLS = 128, 32768
_TOTAL = _ROWS * _COLS
_NC, _NS, _L = 2, 16, 16
_NW = _NC * _NS
_SC_ROWS = 64
_TC_ROWS = _ROWS - _SC_ROWS
_PER_W = _SC_ROWS * _COLS // _NW
_CHUNK = 16384
_NCHUNK = _PER_W // _CHUNK
_U = 8
_TCB = 8

_mesh = plsc.VectorSubcoreMesh(
    core_axis_name="c", subcore_axis_name="s", num_cores=_NC, num_subcores=_NS
)


@functools.partial(
    pl.kernel,
    out_type=jax.ShapeDtypeStruct((_NW, 3 * _L), jnp.float32),
    mesh=_mesh,
    scratch_types=[
        pltpu.VMEM((3 * _CHUNK,), jnp.float32),
        pltpu.VMEM((3 * _CHUNK,), jnp.int32),
        pltpu.VMEM((3 * _L,), jnp.float32),
        pltpu.SemaphoreType.DMA,
        pltpu.SemaphoreType.DMA,
        pltpu.SemaphoreType.DMA,
    ],
)
def _partial_sums(x_hbm, lbl_hbm, out_hbm, xb, lb, accv, sem0, sem1, sem2):
    wid = lax.axis_index("s") * _NC + lax.axis_index("c")
    rows_per_w = _SC_ROWS // _NW
    chunks_per_row = _COLS // _CHUNK
    row0 = wid * rows_per_w
    sems = (sem0, sem1, sem2)
    hx = [None, None]
    hl = [None, None]

    def start(i):
        b = i % 3
        row = row0 + i // chunks_per_row
        col = (i % chunks_per_row) * _CHUNK
        hx[b] = pltpu.async_copy(
            x_hbm.at[row, pl.ds(col, _CHUNK)], xb.at[pl.ds(b * _CHUNK, _CHUNK)], sems[b]
        )
        hl[b] = pltpu.async_copy(
            lbl_hbm.at[row, pl.ds(col, _CHUNK)], lb.at[pl.ds(b * _CHUNK, _CHUNK)], sems[b]
        )

    _P = _L * _U

    def accumulate(xv, lv, accs, next_chunks):
        nxt = []
        for nc in next_chunks:
            nxt.append(
                (nc % 3, row0 + nc // chunks_per_row, (nc % chunks_per_row) * _CHUNK)
            )

        @plsc.parallel_loop(0, _CHUNK, step=_P, carry=accs)
        def accs(i, accs):
            for nb, nrow, ncol in nxt:
                pltpu.async_copy(
                    x_hbm.at[nrow, pl.ds(ncol + i, _P)],
                    xb.at[pl.ds(nb * _CHUNK + i, _P)],
                    sems[nb],
                )
                pltpu.async_copy(
                    lbl_hbm.at[nrow, pl.ds(ncol + i, _P)],
                    lb.at[pl.ds(nb * _CHUNK + i, _P)],
                    sems[nb],
                )
            accs = list(accs)
            for u in range(_U):
                a, p, c = accs[3 * (u % 2) : 3 * (u % 2) + 3]
                sl = pl.ds(i + u * _L, _L)
                lbl = lv[sl]
                sig = 1.0 / (1.0 + jnp.exp(xv[sl] * -1.0))
                accs[3 * (u % 2) : 3 * (u % 2) + 3] = [
                    a + sig,
                    p + jnp.where(lbl != 0, sig, 0.0),
                    c + lbl,
                ]
            return tuple(accs)

        return accs

    def drain(b):
        pltpu.make_async_copy(x_hbm.at[row0, pl.ds(0, _CHUNK)], xb.at[pl.ds(b * _CHUNK, _CHUNK)], sems[b]).wait()
        pltpu.make_async_copy(lbl_hbm.at[row0, pl.ds(0, _CHUNK)], lb.at[pl.ds(b * _CHUNK, _CHUNK)], sems[b]).wait()

    start(0)
    zf = jnp.zeros((_L,), jnp.float32)
    zi = jnp.zeros((_L,), jnp.int32)
    accs = (zf, zf, zi, zf, zf, zi)
    for i in range(_NCHUNK):
        b = i % 3
        if i == 0:
            hx[0].wait()
            hl[0].wait()
        else:
            drain(b)
        if i == 0:
            nxt = [c for c in (1, 2) if c < _NCHUNK]
        else:
            nxt = [i + 2] if i + 2 < _NCHUNK else []
        accs = accumulate(xb.at[pl.ds(b * _CHUNK, _CHUNK)], lb.at[pl.ds(b * _CHUNK, _CHUNK)], accs, nxt)

    accv[pl.ds(0, _L)] = accs[0] + accs[3]
    accv[pl.ds(_L, _L)] = accs[1] + accs[4]
    accv[pl.ds(2 * _L, _L)] = (accs[2] + accs[5]).astype(jnp.float32)
    pltpu.sync_copy(accv, out_hbm.at[wid])


def _tc_partial_body(x_ref, l_ref, o_ref):
    x = x_ref[...]
    lf = l_ref[...].astype(jnp.float32)
    sig = 1.0 / (1.0 + jnp.exp(-x))
    sa = jnp.sum(sig.reshape(_TCB, _COLS // 128, 128), axis=1)
    sp = jnp.sum((sig * lf).reshape(_TCB, _COLS // 128, 128), axis=1)
    cn = jnp.sum(lf.reshape(_TCB, _COLS // 128, 128), axis=1)
    part = jnp.stack([sa, sp, cn])

    @pl.when(pl.program_id(0) == 0)
    def _():
        o_ref[...] = part

    @pl.when(pl.program_id(0) != 0)
    def _():
        o_ref[...] += part


_tc_partial = pl.pallas_call(
    _tc_partial_body,
    grid=(_TC_ROWS // _TCB,),
    in_specs=[
        pl.BlockSpec((_TCB, _COLS), lambda i: (i + _SC_ROWS // _TCB, 0)),
        pl.BlockSpec((_TCB, _COLS), lambda i: (i + _SC_ROWS // _TCB, 0)),
    ],
    out_specs=pl.BlockSpec((3, _TCB, 128), lambda i: (0, 0, 0)),
    out_shape=jax.ShapeDtypeStruct((3, _TCB, 128), jnp.float32),
)


def _finalize_body(p_ref, t_ref, o_ref):
    p = p_ref[...]
    t = t_ref[...]
    s_all = jnp.sum(p[:, 0:_L]) + jnp.sum(t[0])
    s_pos = jnp.sum(p[:, _L : 2 * _L]) + jnp.sum(t[1])
    k_pos = jnp.sum(p[:, 2 * _L : 3 * _L]) + jnp.sum(t[2])
    k_neg = _TOTAL - k_pos
    diff = (s_all - s_pos) / k_neg - s_pos / k_pos
    o_ref[...] = jnp.maximum(diff, 0.0)[None, None]


_finalize = pl.pallas_call(
    _finalize_body,
    out_shape=jax.ShapeDtypeStruct((1, 1), jnp.float32),
)


@jax.jit
def kernel(outputs, labels):
    partials = _partial_sums(outputs, labels)
    tc_partials = _tc_partial(outputs, labels)
    return _finalize(partials, tc_partials)[0, 0]


# speedup vs baseline: 1.0181x; 1.0181x over previous
import functools

import jax
import jax.numpy as jnp
from jax import lax
from jax.experimental import pallas as pl
from jax.experimental.pallas import tpu as pltpu
from jax.experimental.pallas import tpu_sc as plsc

_ROWS, _COLS = 128, 32768
_TOTAL = _ROWS * _COLS
_NC, _NS, _L = 2, 16, 16
_NW = _NC * _NS
_SC_ROWS = 64
_TC_ROWS = _ROWS - _SC_ROWS
_PER_W = _SC_ROWS * _COLS // _NW
_CHUNK = 16384
_NCHUNK = _PER_W // _CHUNK
_U = 8
_TCB = 8

_mesh = plsc.VectorSubcoreMesh(
    core_axis_name="c", subcore_axis_name="s", num_cores=_NC, num_subcores=_NS
)


@functools.partial(
    pl.kernel,
    out_type=jax.ShapeDtypeStruct((_NW, 3 * _L), jnp.float32),
    mesh=_mesh,
    scratch_types=[
        pltpu.VMEM((3 * _CHUNK,), jnp.float32),
        pltpu.VMEM((3 * _CHUNK,), jnp.int32),
        pltpu.VMEM((3 * _L,), jnp.float32),
        pltpu.SemaphoreType.DMA,
        pltpu.SemaphoreType.DMA,
        pltpu.SemaphoreType.DMA,
    ],
)
def _partial_sums(x_hbm, lbl_hbm, out_hbm, xb, lb, accv, sem0, sem1, sem2):
    wid = lax.axis_index("s") * _NC + lax.axis_index("c")
    rows_per_w = _SC_ROWS // _NW
    chunks_per_row = _COLS // _CHUNK
    row0 = wid * rows_per_w
    sems = (sem0, sem1, sem2)
    hx = [None, None]
    hl = [None, None]

    def start(i):
        b = i % 3
        row = row0 + i // chunks_per_row
        col = (i % chunks_per_row) * _CHUNK
        hx[b] = pltpu.async_copy(
            x_hbm.at[row, pl.ds(col, _CHUNK)], xb.at[pl.ds(b * _CHUNK, _CHUNK)], sems[b]
        )
        hl[b] = pltpu.async_copy(
            lbl_hbm.at[row, pl.ds(col, _CHUNK)], lb.at[pl.ds(b * _CHUNK, _CHUNK)], sems[b]
        )

    _P = _L * _U

    def accumulate(xv, lv, accs, next_chunks):
        nxt = []
        for nc in next_chunks:
            nxt.append(
                (nc % 3, row0 + nc // chunks_per_row, (nc % chunks_per_row) * _CHUNK)
            )

        @plsc.parallel_loop(0, _CHUNK, step=_P, carry=accs)
        def accs(i, accs):
            for nb, nrow, ncol in nxt:
                pltpu.async_copy(
                    x_hbm.at[nrow, pl.ds(ncol + i, _P)],
                    xb.at[pl.ds(nb * _CHUNK + i, _P)],
                    sems[nb],
                )
                pltpu.async_copy(
                    lbl_hbm.at[nrow, pl.ds(ncol + i, _P)],
                    lb.at[pl.ds(nb * _CHUNK + i, _P)],
                    sems[nb],
                )
            accs = list(accs)
            for u in range(_U):
                a, p, c = accs[3 * (u % 2) : 3 * (u % 2) + 3]
                sl = pl.ds(i + u * _L, _L)
                lbl = lv[sl]
                sig = 1.0 / (1.0 + jnp.exp(xv[sl] * -1.0))
                accs[3 * (u % 2) : 3 * (u % 2) + 3] = [
                    a + sig,
                    p + jnp.where(lbl != 0, sig, 0.0),
                    c + lbl,
                ]
            return tuple(accs)

        return accs

    def drain(b):
        pltpu.make_async_copy(x_hbm.at[row0, pl.ds(0, _CHUNK)], xb.at[pl.ds(b * _CHUNK, _CHUNK)], sems[b]).wait()
        pltpu.make_async_copy(lbl_hbm.at[row0, pl.ds(0, _CHUNK)], lb.at[pl.ds(b * _CHUNK, _CHUNK)], sems[b]).wait()

    start(0)
    start(1)
    zf = jnp.zeros((_L,), jnp.float32)
    zi = jnp.zeros((_L,), jnp.int32)
    accs = (zf, zf, zi, zf, zf, zi)
    for i in range(_NCHUNK):
        b = i % 3
        if i < 2:
            hx[i].wait()
            hl[i].wait()
        else:
            drain(b)
        nxt = [i + 2] if i + 2 < _NCHUNK else []
        accs = accumulate(xb.at[pl.ds(b * _CHUNK, _CHUNK)], lb.at[pl.ds(b * _CHUNK, _CHUNK)], accs, nxt)

    accv[pl.ds(0, _L)] = accs[0] + accs[3]
    accv[pl.ds(_L, _L)] = accs[1] + accs[4]
    accv[pl.ds(2 * _L, _L)] = (accs[2] + accs[5]).astype(jnp.float32)
    pltpu.sync_copy(accv, out_hbm.at[wid])


def _tc_partial_body(x_ref, l_ref, o_ref):
    x = x_ref[...]
    lf = l_ref[...].astype(jnp.float32)
    sig = 1.0 / (1.0 + jnp.exp(-x))
    sa = jnp.sum(sig.reshape(_TCB, _COLS // 128, 128), axis=1)
    sp = jnp.sum((sig * lf).reshape(_TCB, _COLS // 128, 128), axis=1)
    cn = jnp.sum(lf.reshape(_TCB, _COLS // 128, 128), axis=1)
    part = jnp.stack([sa, sp, cn])

    @pl.when(pl.program_id(0) == 0)
    def _():
        o_ref[...] = part

    @pl.when(pl.program_id(0) != 0)
    def _():
        o_ref[...] += part


_tc_partial = pl.pallas_call(
    _tc_partial_body,
    grid=(_TC_ROWS // _TCB,),
    in_specs=[
        pl.BlockSpec((_TCB, _COLS), lambda i: (i + _SC_ROWS // _TCB, 0)),
        pl.BlockSpec((_TCB, _COLS), lambda i: (i + _SC_ROWS // _TCB, 0)),
    ],
    out_specs=pl.BlockSpec((3, _TCB, 128), lambda i: (0, 0, 0)),
    out_shape=jax.ShapeDtypeStruct((3, _TCB, 128), jnp.float32),
)


def _finalize_body(p_ref, t_ref, o_ref):
    p = p_ref[...]
    t = t_ref[...]
    s_all = jnp.sum(p[:, 0:_L]) + jnp.sum(t[0])
    s_pos = jnp.sum(p[:, _L : 2 * _L]) + jnp.sum(t[1])
    k_pos = jnp.sum(p[:, 2 * _L : 3 * _L]) + jnp.sum(t[2])
    k_neg = _TOTAL - k_pos
    diff = (s_all - s_pos) / k_neg - s_pos / k_pos
    o_ref[...] = jnp.maximum(diff, 0.0)[None, None]


_finalize = pl.pallas_call(
    _finalize_body,
    out_shape=jax.ShapeDtypeStruct((1, 1), jnp.float32),
)


@jax.jit
def kernel(outputs, labels):
    partials = _partial_sums(outputs, labels)
    tc_partials = _tc_partial(outputs, labels)
    return _finalize(partials, tc_partials)[0, 0]
